# baseline (speedup 1.0000x reference)
import functools
import jax
import jax.numpy as jnp
from jax import lax
from jax.experimental import pallas as pl
from jax.experimental.pallas import tpu as pltpu
from jax.experimental.pallas import tpu_sc as plsc

_B = 16384
_D = 64
_MESH = plsc.VectorSubcoreMesh(
    core_axis_name="c", subcore_axis_name="s", num_cores=1)


def _body(mem_h, val_h, out_h, buf, sem):
    wid = lax.axis_index("s")
    pltpu.sync_copy(val_h.at[pl.ds(wid * 1024, 1024)], buf)
    pltpu.sync_copy(buf, out_h.at[pl.ds(wid * 1024, 1024)])


_triv = functools.partial(
    pl.kernel,
    out_type=jax.ShapeDtypeStruct((_B, _D), jnp.float32),
    mesh=_MESH,
    scratch_types=[
        pltpu.VMEM((1024, _D), jnp.float32),
        pltpu.SemaphoreType.DMA,
    ],
)(_body)


def kernel(mem, idx, val):
    return _triv(mem, val)
